# Initial kernel scaffold; baseline (speedup 1.0000x reference)
import jax
import jax.numpy as jnp
from jax.experimental import pallas as pl


def kernel(residue_idx, embed_weight):
    raise NotImplementedError("write your pallas kernel here")


# speedup vs baseline: 5.8845x; 5.8845x over previous
import functools

import jax
import jax.numpy as jnp
from jax import lax
from jax.experimental import pallas as pl
from jax.experimental.pallas import tpu as pltpu
from jax.experimental.pallas import tpu_sc as plsc

MAX_LEN = 3000
EMBED_DIM = 32
BATCH = 16384
SEQ = 200

TOTAL = BATCH * SEQ
NW = 32
PER_W = TOTAL // NW
IDXV = 128
K = 8
CHUNK = K * IDXV
NCHUNK = PER_W // CHUNK
NBUF = 2
NITER = NCHUNK // NBUF

assert PER_W % CHUNK == 0 and NCHUNK % NBUF == 0


def _emb_body(idx_hbm, tab_hbm, out_hbm,
              idx_v, rows_v,
              sem_i0, sem_i1, sem_g0, sem_g1, sem_o0, sem_o1):
    sem_i = [sem_i0, sem_i1]
    sem_g = [sem_g0, sem_g1]
    sem_o = [sem_o0, sem_o1]

    wid = lax.axis_index("s") * 2 + lax.axis_index("c")
    w_row0 = wid * (PER_W // IDXV)
    w_out0 = wid * PER_W

    def stage_idx(chunk, b):
        src = idx_hbm.at[pl.ds(w_row0 + chunk * K, K)]
        pltpu.async_copy(src, idx_v.at[b], sem_i[b])

    for b in range(NBUF):
        stage_idx(b, b)

    def loop_body(i, carry):
        for b in range(NBUF):
            chunk = i * NBUF + b
            pltpu.make_async_copy(idx_hbm.at[pl.ds(0, K)], idx_v.at[b],
                                  sem_i[b]).wait()

            @pl.when(i >= 1)
            def _wait_prev_out():
                pltpu.make_async_copy(rows_v.at[b],
                                      out_hbm.at[pl.ds(0, CHUNK)],
                                      sem_o[b]).wait()

            for j in range(K):
                pltpu.async_copy(tab_hbm.at[idx_v.at[b, j]],
                                 rows_v.at[b, pl.ds(j * IDXV, IDXV), :],
                                 sem_g[b])
            for j in range(K):
                pltpu.make_async_copy(tab_hbm.at[idx_v.at[b, j]],
                                      rows_v.at[b, pl.ds(j * IDXV, IDXV), :],
                                      sem_g[b]).wait()

            @pl.when(i < NITER - 1)
            def _stage_next():
                stage_idx(chunk + NBUF, b)

            pltpu.async_copy(rows_v.at[b],
                             out_hbm.at[pl.ds(w_out0 + chunk * CHUNK, CHUNK)],
                             sem_o[b])
        return carry

    lax.fori_loop(0, NITER, loop_body, 0)

    for b in range(NBUF):
        pltpu.make_async_copy(rows_v.at[b], out_hbm.at[pl.ds(0, CHUNK)],
                              sem_o[b]).wait()


@jax.jit
def _emb(idx128, table):
    mesh = plsc.VectorSubcoreMesh(core_axis_name="c", subcore_axis_name="s")
    f = functools.partial(
        pl.kernel,
        mesh=mesh,
        out_type=jax.ShapeDtypeStruct((TOTAL, EMBED_DIM), jnp.float32),
        scratch_types=[
            pltpu.VMEM((NBUF, K, IDXV), jnp.int32),
            pltpu.VMEM((NBUF, CHUNK, EMBED_DIM), jnp.float32),
            pltpu.SemaphoreType.DMA,
            pltpu.SemaphoreType.DMA,
            pltpu.SemaphoreType.DMA,
            pltpu.SemaphoreType.DMA,
            pltpu.SemaphoreType.DMA,
            pltpu.SemaphoreType.DMA,
        ],
        compiler_params=pltpu.CompilerParams(use_tc_tiling_on_sc=False),
    )(_emb_body)
    return f(idx128, table)


def kernel(residue_idx, embed_weight):
    idx = residue_idx.astype(jnp.int32).reshape(TOTAL // IDXV, IDXV)
    out = _emb(idx, embed_weight)
    return out.reshape(BATCH, SEQ, EMBED_DIM)
